# baseline (speedup 1.0000x reference)
import functools

import jax
import jax.numpy as jnp
from jax import lax
from jax.experimental import pallas as pl
from jax.experimental.pallas import tpu as pltpu
from jax.experimental.pallas import tpu_sc as plsc

_B, _N, _K, _F = 4, 4096, 16, 256
_QB = 512
_NBLK = _N // _QB
_INF = float("inf")
_RECIP15 = 0.06666667014360427856


def _foldhalf(vals):
    vals = list(vals)
    while len(vals) > 1:
        h = len(vals) // 2
        vals = [vals[i] + vals[i + h] for i in range(h)]
    return vals[0]


def _tree(vals):
    vals = list(vals)
    while len(vals) > 1:
        vals = [vals[2 * i] + vals[2 * i + 1] for i in range(len(vals) // 2)]
    return vals[0]


def _bf(x):
    return x.astype(jnp.bfloat16).astype(jnp.float32)


def _nrmz(x):
    return x / jnp.clip(jnp.linalg.norm(x, axis=-1, keepdims=True), 1e-12)


def _knn1_body(q_ref, pT_ref, idx_ref, mean_ref, cov_ref):
    b = pl.program_id(0)
    q = q_ref[0]
    pT = pT_ref[0]
    qx, qy, qz = q[:, 0:1], q[:, 1:2], q[:, 2:3]
    qq = (qx * qx + qy * qy) + qz * qz
    px, py, pz = pT[0:1, :], pT[1:2, :], pT[2:3, :]
    pp = (px * px + py * py) + pz * pz
    e = jax.lax.dot_general(q, pT, (((1,), (0,)), ((), ())),
                            preferred_element_type=jnp.float32)
    d2 = (qq + pp) - 2.0 * e
    lane = jax.lax.broadcasted_iota(jnp.int32, (_QB, _N), 1)
    xs, ys, zs, ids = [], [], [], []
    for _ in range(_K):
        m = jnp.min(d2, axis=1, keepdims=True)
        idx = jnp.min(jnp.where(d2 == m, lane, _N), axis=1, keepdims=True)
        sel = lane == idx
        xs.append(jnp.sum(jnp.where(sel, px, 0.0), axis=1, keepdims=True))
        ys.append(jnp.sum(jnp.where(sel, py, 0.0), axis=1, keepdims=True))
        zs.append(jnp.sum(jnp.where(sel, pz, 0.0), axis=1, keepdims=True))
        ids.append(idx)
        d2 = jnp.where(sel, _INF, d2)
    mx = _foldhalf(xs) / 16.0
    my = _foldhalf(ys) / 16.0
    mz = _foldhalf(zs) / 16.0
    cx = [_bf(v - mx) for v in xs]
    cy = [_bf(v - my) for v in ys]
    cz = [_bf(v - mz) for v in zs]

    def _cov(a, b_):
        return _tree([a[k] * b_[k] for k in range(_K)]) * _RECIP15

    idx_ref[0] = jnp.concatenate(ids, axis=1) + b * _N
    mean_ref[0] = jnp.concatenate([mx, my, mz], axis=1)
    cov_ref[0] = jnp.concatenate(
        [_cov(cx, cx), _cov(cx, cy), _cov(cx, cz),
         _cov(cy, cy), _cov(cy, cz), _cov(cz, cz)], axis=1)


def _sc_gather(table, idx):
    info = plsc.get_sparse_core_info()
    nw = info.num_cores * info.num_subcores
    m = idx.shape[0]
    b_per_w = m // nw
    chunk = 256
    n_chunks = b_per_w // chunk
    mesh = plsc.VectorSubcoreMesh(core_axis_name="c", subcore_axis_name="s")

    @functools.partial(
        pl.kernel, mesh=mesh,
        out_type=jax.ShapeDtypeStruct((m, 128), jnp.float32),
        scratch_types=[
            pltpu.VMEM((chunk,), jnp.int32),
            pltpu.VMEM((chunk, 128), jnp.float32),
            pltpu.SemaphoreType.DMA,
        ],
    )
    def k(table_hbm, idx_hbm, out_hbm, idx_v, rows_v, sem):
        wid = lax.axis_index("s") * info.num_cores + lax.axis_index("c")
        base = wid * b_per_w
        for c in range(n_chunks):
            off = base + c * chunk
            pltpu.sync_copy(idx_hbm.at[pl.ds(off, chunk)], idx_v)
            pltpu.async_copy(table_hbm.at[idx_v], rows_v, sem).wait()
            pltpu.sync_copy(rows_v, out_hbm.at[pl.ds(off, chunk)])

    return k(table, idx)


def _nnmean_body(g_ref, o_ref):
    g = g_ref[0]
    parts = [g[:, 128 * k:128 * k + 3] for k in range(_K)]
    o_ref[0] = _foldhalf(parts) / 16.0


def _fps_body(xyzT_ref, m1T_ref, mnT_ref, nuT_ref, fpsT_ref):
    X0 = xyzT_ref[:, 0, :]
    Y0 = xyzT_ref[:, 1, :]
    Z0 = xyzT_ref[:, 2, :]
    dx = X0 - m1T_ref[:, 0, :]
    dy = Y0 - m1T_ref[:, 1, :]
    dz = Z0 - m1T_ref[:, 2, :]
    nx = mnT_ref[:, 0, :]
    ny = mnT_ref[:, 1, :]
    nz = mnT_ref[:, 2, :]
    p00, p01, p02 = _bf(nx * nx), _bf(nx * ny), _bf(nx * nz)
    p11, p12, p22 = _bf(ny * ny), _bf(ny * nz), _bf(nz * nz)
    bx, by, bz = _bf(dx), _bf(dy), _bf(dz)
    X = X0 - ((p00 * bx + p01 * by) + p02 * bz)
    Y = Y0 - ((p01 * bx + p11 * by) + p12 * bz)
    Z = Z0 - ((p02 * bx + p12 * by) + p22 * bz)
    nuT_ref[:, 0, :] = X
    nuT_ref[:, 1, :] = Y
    nuT_ref[:, 2, :] = Z

    lane = jax.lax.broadcasted_iota(jnp.int32, (_B, _N), 1)
    lane_f = jax.lax.broadcasted_iota(jnp.int32, (_B, _F), 1)
    lx0, ly0, lz0 = X[:, 0:1], Y[:, 0:1], Z[:, 0:1]
    ax, ay, az = X - lx0, Y - ly0, Z - lz0
    dists = (ax * ax + ay * ay) + az * az
    fx = jnp.where(lane_f == 0, lx0, 0.0)
    fy = jnp.where(lane_f == 0, ly0, 0.0)
    fz = jnp.where(lane_f == 0, lz0, 0.0)

    def body(i, carry):
        dists, fx, fy, fz = carry
        m = jnp.max(dists, axis=1, keepdims=True)
        nxt = jnp.min(jnp.where(dists == m, lane, _N), axis=1, keepdims=True)
        sel = lane == nxt
        lx = jnp.sum(jnp.where(sel, X, 0.0), axis=1, keepdims=True)
        ly = jnp.sum(jnp.where(sel, Y, 0.0), axis=1, keepdims=True)
        lz = jnp.sum(jnp.where(sel, Z, 0.0), axis=1, keepdims=True)
        fx = fx + jnp.where(lane_f == i, lx, 0.0)
        fy = fy + jnp.where(lane_f == i, ly, 0.0)
        fz = fz + jnp.where(lane_f == i, lz, 0.0)
        ax, ay, az = X - lx, Y - ly, Z - lz
        d = (ax * ax + ay * ay) + az * az
        return jnp.minimum(dists, d), fx, fy, fz

    _, fx, fy, fz = jax.lax.fori_loop(1, _F, body, (dists, fx, fy, fz))
    fpsT_ref[:, 0, :] = fx
    fpsT_ref[:, 1, :] = fy
    fpsT_ref[:, 2, :] = fz


def _knn2_body(q_ref, pT_ref, mT_ref, lm_ref, nn_ref):
    q = q_ref[0]
    pT = pT_ref[0]
    mT = mT_ref[0]
    qx, qy, qz = q[:, 0:1], q[:, 1:2], q[:, 2:3]
    qq = (qx * qx + qy * qy) + qz * qz
    px, py, pz = pT[0:1, :], pT[1:2, :], pT[2:3, :]
    pp = (px * px + py * py) + pz * pz
    e = jax.lax.dot_general(q, pT, (((1,), (0,)), ((), ())),
                            preferred_element_type=jnp.float32)
    d2 = (qq + pp) - 2.0 * e
    nxr, nyr, nzr = mT[0:1, :], mT[1:2, :], mT[2:3, :]
    lane = jax.lax.broadcasted_iota(jnp.int32, (_F, _N), 1)
    exs, eys, ezs, nxs, nys, nzs = [], [], [], [], [], []
    for _ in range(_K):
        m = jnp.min(d2, axis=1, keepdims=True)
        idx = jnp.min(jnp.where(d2 == m, lane, _N), axis=1, keepdims=True)
        sel = lane == idx
        exs.append(jnp.sum(jnp.where(sel, px, 0.0), axis=1, keepdims=True))
        eys.append(jnp.sum(jnp.where(sel, py, 0.0), axis=1, keepdims=True))
        ezs.append(jnp.sum(jnp.where(sel, pz, 0.0), axis=1, keepdims=True))
        nxs.append(jnp.sum(jnp.where(sel, nxr, 0.0), axis=1, keepdims=True))
        nys.append(jnp.sum(jnp.where(sel, nyr, 0.0), axis=1, keepdims=True))
        nzs.append(jnp.sum(jnp.where(sel, nzr, 0.0), axis=1, keepdims=True))
        d2 = jnp.where(sel, _INF, d2)
    lm_ref[0] = jnp.concatenate(
        [_foldhalf(exs) / 16.0, _foldhalf(eys) / 16.0, _foldhalf(ezs) / 16.0],
        axis=1)
    nn_ref[0] = jnp.concatenate(
        [_foldhalf(nxs) / 16.0, _foldhalf(nys) / 16.0, _foldhalf(nzs) / 16.0],
        axis=1)


def _smooth_body(q_ref, lm_ref, nf_ref, o_ref):
    q = q_ref[0]
    lm = lm_ref[0]
    nf = nf_ref[0]
    qx, qy, qz = q[:, 0:1], q[:, 1:2], q[:, 2:3]
    nfx, nfy, nfz = nf[:, 0:1], nf[:, 1:2], nf[:, 2:3]
    dfx = qx - lm[:, 0:1]
    dfy = qy - lm[:, 1:2]
    dfz = qz - lm[:, 2:3]
    t00, t01, t02 = _bf(1.0 - nfx * nfx), _bf(0.0 - nfx * nfy), _bf(0.0 - nfx * nfz)
    t10, t11, t12 = _bf(0.0 - nfy * nfx), _bf(1.0 - nfy * nfy), _bf(0.0 - nfy * nfz)
    t20, t21, t22 = _bf(0.0 - nfz * nfx), _bf(0.0 - nfz * nfy), _bf(1.0 - nfz * nfz)
    bx, by, bz = _bf(dfx), _bf(dfy), _bf(dfz)
    dtx = (t00 * bx + t01 * by) + t02 * bz
    dty = (t10 * bx + t11 * by) + t12 * bz
    dtz = (t20 * bx + t21 * by) + t22 * bz
    o_ref[0] = jnp.concatenate([qx - dtx, qy - dty, qz - dtz], axis=1)


def kernel(xyz):
    xyzT = jnp.transpose(xyz, (0, 2, 1))

    idx, mean1, cov6 = pl.pallas_call(
        _knn1_body,
        grid=(_B, _NBLK),
        in_specs=[
            pl.BlockSpec((1, _QB, 3), lambda b, i: (b, i, 0)),
            pl.BlockSpec((1, 3, _N), lambda b, i: (b, 0, 0)),
        ],
        out_specs=[
            pl.BlockSpec((1, _QB, _K), lambda b, i: (b, i, 0)),
            pl.BlockSpec((1, _QB, 3), lambda b, i: (b, i, 0)),
            pl.BlockSpec((1, _QB, 6), lambda b, i: (b, i, 0)),
        ],
        out_shape=[
            jax.ShapeDtypeStruct((_B, _N, _K), jnp.int32),
            jax.ShapeDtypeStruct((_B, _N, 3), jnp.float32),
            jax.ShapeDtypeStruct((_B, _N, 6), jnp.float32),
        ],
    )(xyz, xyzT)

    c = cov6
    cov = jnp.stack(
        [c[..., 0], c[..., 1], c[..., 2],
         c[..., 1], c[..., 3], c[..., 4],
         c[..., 2], c[..., 4], c[..., 5]], axis=-1).reshape(_B, _N, 3, 3)
    _, evec = jnp.linalg.eigh(cov)
    normals = _nrmz(evec[..., :, 0])

    table = jnp.pad(normals.reshape(_B * _N, 3), ((0, 0), (0, 125)))
    gathered = _sc_gather(table, idx.reshape(_B * _N * _K))
    gathered = gathered.reshape(_B, _N, _K * 128)

    nn_mean = pl.pallas_call(
        _nnmean_body,
        grid=(_B, _NBLK),
        in_specs=[pl.BlockSpec((1, _QB, _K * 128), lambda b, i: (b, i, 0))],
        out_specs=pl.BlockSpec((1, _QB, 3), lambda b, i: (b, i, 0)),
        out_shape=jax.ShapeDtypeStruct((_B, _N, 3), jnp.float32),
    )(gathered)
    mean_normal = _nrmz(nn_mean)

    m1T = jnp.transpose(mean1, (0, 2, 1))
    mnT = jnp.transpose(mean_normal, (0, 2, 1))
    nuT, fpsT = pl.pallas_call(
        _fps_body,
        out_shape=[
            jax.ShapeDtypeStruct((_B, 3, _N), jnp.float32),
            jax.ShapeDtypeStruct((_B, 3, _F), jnp.float32),
        ],
    )(xyzT, m1T, mnT)
    xyz_fps = jnp.transpose(fpsT, (0, 2, 1))

    lm_fps, nnf = pl.pallas_call(
        _knn2_body,
        grid=(_B,),
        in_specs=[
            pl.BlockSpec((1, _F, 3), lambda b: (b, 0, 0)),
            pl.BlockSpec((1, 3, _N), lambda b: (b, 0, 0)),
            pl.BlockSpec((1, 3, _N), lambda b: (b, 0, 0)),
        ],
        out_specs=[
            pl.BlockSpec((1, _F, 3), lambda b: (b, 0, 0)),
            pl.BlockSpec((1, _F, 3), lambda b: (b, 0, 0)),
        ],
        out_shape=[
            jax.ShapeDtypeStruct((_B, _F, 3), jnp.float32),
            jax.ShapeDtypeStruct((_B, _F, 3), jnp.float32),
        ],
    )(xyz_fps, nuT, mnT)
    nf = _nrmz(nnf)

    out = pl.pallas_call(
        _smooth_body,
        grid=(_B,),
        in_specs=[
            pl.BlockSpec((1, _F, 3), lambda b: (b, 0, 0)),
            pl.BlockSpec((1, _F, 3), lambda b: (b, 0, 0)),
            pl.BlockSpec((1, _F, 3), lambda b: (b, 0, 0)),
        ],
        out_specs=pl.BlockSpec((1, _F, 3), lambda b: (b, 0, 0)),
        out_shape=jax.ShapeDtypeStruct((_B, _F, 3), jnp.float32),
    )(xyz_fps, lm_fps, nf)
    return out
